# baseline (speedup 1.0000x reference)
import jax
import jax.numpy as jnp
from jax.experimental import pallas as pl
from jax.experimental.pallas import tpu as pltpu

_F32 = jnp.float32
_BF16 = jnp.bfloat16


def _conv1x1_ksplit_kernel(w_ref, xlo_ref, xhi_ref, o_ref):
    kh = xlo_ref.shape[1]
    lo = jnp.dot(w_ref[:, :kh], xlo_ref[0].astype(_BF16),
                 preferred_element_type=_F32)
    hi = jnp.dot(w_ref[:, kh:], xhi_ref[0].astype(_BF16),
                 preferred_element_type=_F32)
    o_ref[0] = lo + hi


def _pick_tile(hw, cap=2048):
    if hw % 128 != 0:
        return hw
    for t in range(min(cap, hw), 127, -128):
        if hw % t == 0:
            return t
    return hw


def kernel(x_nchw, w2d):
    N, C, H, W = x_nchw.shape
    COUT, CIN = w2d.shape
    HW = H * W
    x3d = x_nchw.reshape(N, CIN, HW)
    wb = w2d.astype(_BF16)

    tile = _pick_tile(HW)
    grid = (N, HW // tile)
    kh = CIN // 2

    out3d = pl.pallas_call(
        _conv1x1_ksplit_kernel,
        out_shape=jax.ShapeDtypeStruct((N, COUT, HW), _F32),
        grid=grid,
        in_specs=[
            pl.BlockSpec((COUT, CIN), lambda n, s: (0, 0)),
            pl.BlockSpec((1, kh, tile), lambda n, s: (n, 0, s)),
            pl.BlockSpec((1, kh, tile), lambda n, s: (n, 1, s)),
        ],
        out_specs=pl.BlockSpec((1, COUT, tile), lambda n, s: (n, 0, s)),
        compiler_params=pltpu.CompilerParams(
            dimension_semantics=("parallel", "parallel"),
            vmem_limit_bytes=100 * 2**20,
        ),
        cost_estimate=pl.CostEstimate(
            flops=2 * N * HW * CIN * COUT,
            transcendentals=0,
            bytes_accessed=(N * CIN * HW + COUT * CIN + N * COUT * HW) * 4,
        ),
    )(wb, x3d, x3d)
    return out3d.reshape(N, COUT, H, W)


# speedup vs baseline: 1.0049x; 1.0049x over previous
import jax
import jax.numpy as jnp
from jax.experimental import pallas as pl
from jax.experimental.pallas import tpu as pltpu

_F32 = jnp.float32
_BF16 = jnp.bfloat16


def _conv1x1_kernel(w_ref, x_ref, o_ref):
    wb = w_ref[...].astype(_BF16)
    xb = x_ref[0].astype(_BF16)
    o_ref[0] = jnp.dot(wb, xb, preferred_element_type=_F32)


def _pick_tile(hw, cap=2048):
    if hw % 128 != 0:
        return hw
    for t in range(min(cap, hw), 127, -128):
        if hw % t == 0:
            return t
    return hw


def kernel(x_nchw, w2d):
    N, C, H, W = x_nchw.shape
    COUT, CIN = w2d.shape
    HW = H * W
    x3d = x_nchw.reshape(N, CIN, HW)

    tile = _pick_tile(HW)
    s = HW // tile

    x_bytes = CIN * tile * 4
    o_bytes = COUT * tile * 4
    vmem = int(min(112 * 2**20,
                   2 * (x_bytes + o_bytes) + COUT * CIN * 4 + (8 << 20)))

    out3d = pl.pallas_call(
        _conv1x1_kernel,
        out_shape=jax.ShapeDtypeStruct((N, COUT, HW), _F32),
        grid=(N * s,),
        in_specs=[
            pl.BlockSpec((COUT, CIN), lambda i: (0, 0)),
            pl.BlockSpec((1, CIN, tile), lambda i: (i // s, 0, i % s)),
        ],
        out_specs=pl.BlockSpec((1, COUT, tile), lambda i: (i // s, 0, i % s)),
        compiler_params=pltpu.CompilerParams(
            dimension_semantics=("parallel",),
            vmem_limit_bytes=vmem,
        ),
        cost_estimate=pl.CostEstimate(
            flops=2 * N * HW * CIN * COUT,
            transcendentals=0,
            bytes_accessed=(N * CIN * HW + COUT * CIN + N * COUT * HW) * 4,
        ),
    )(w2d, x3d)
    return out3d.reshape(N, COUT, H, W)


# speedup vs baseline: 1.0066x; 1.0016x over previous
import jax
import jax.numpy as jnp
from jax.experimental import pallas as pl
from jax.experimental.pallas import tpu as pltpu

_F32 = jnp.float32
_BF16 = jnp.bfloat16


def _conv1x1_kernel(w_ref, x_ref, o_ref):
    wb = w_ref[...].astype(_BF16)
    xb = x_ref[0].astype(_BF16)
    o_ref[0] = jnp.dot(wb, xb, preferred_element_type=_F32)


def _pick_tile(hw, cap=2048):
    if hw % 128 != 0:
        return hw
    for t in range(min(cap, hw), 127, -128):
        if hw % t == 0:
            return t
    return hw


def kernel(x_nchw, w2d):
    N, C, H, W = x_nchw.shape
    COUT, CIN = w2d.shape
    HW = H * W
    x3d = x_nchw.reshape(N, CIN, HW)

    tile = _pick_tile(HW)
    s = HW // tile

    x_bytes = CIN * tile * 4
    o_bytes = COUT * tile * 4
    vmem = int(min(112 * 2**20,
                   2 * (x_bytes + o_bytes) + COUT * CIN * 4 + (8 << 20)))

    out3d = pl.pallas_call(
        _conv1x1_kernel,
        out_shape=jax.ShapeDtypeStruct((N, COUT, HW), _F32),
        grid=(N, s),
        in_specs=[
            pl.BlockSpec((COUT, CIN), lambda n, t: (0, 0)),
            pl.BlockSpec((1, CIN, tile), lambda n, t: (n, 0, t)),
        ],
        out_specs=pl.BlockSpec((1, COUT, tile), lambda n, t: (n, 0, t)),
        compiler_params=pltpu.CompilerParams(
            dimension_semantics=("parallel", "parallel"),
            vmem_limit_bytes=vmem,
        ),
        cost_estimate=pl.CostEstimate(
            flops=2 * N * HW * CIN * COUT,
            transcendentals=0,
            bytes_accessed=(N * CIN * HW + COUT * CIN + N * COUT * HW) * 4,
        ),
    )(w2d, x3d)
    return out3d.reshape(N, COUT, H, W)


# speedup vs baseline: 1.0082x; 1.0016x over previous
import jax
import jax.numpy as jnp
from jax.experimental import pallas as pl
from jax.experimental.pallas import tpu as pltpu

_F32 = jnp.float32
_BF16 = jnp.bfloat16


def _conv1x1_kernel(w_ref, x_ref, o_ref):
    wb = w_ref[...].astype(_BF16)
    xb = x_ref[0].astype(_BF16)
    o_ref[0] = jnp.dot(wb, xb, preferred_element_type=_F32)


def _pick_tile(hw, cap=4096):
    if hw % 128 != 0:
        return hw
    for t in range(min(cap, hw), 127, -128):
        if hw % t == 0:
            return t
    return hw


def kernel(x_nchw, w2d):
    N, C, H, W = x_nchw.shape
    COUT, CIN = w2d.shape
    HW = H * W
    x3d = x_nchw.reshape(N, CIN, HW)

    tile = _pick_tile(HW)
    s = HW // tile

    x_bytes = CIN * tile * 4
    o_bytes = COUT * tile * 4
    vmem = int(min(112 * 2**20,
                   2 * (x_bytes + o_bytes) + COUT * CIN * 4 + (8 << 20)))

    out3d = pl.pallas_call(
        _conv1x1_kernel,
        out_shape=jax.ShapeDtypeStruct((N, COUT, HW), _F32),
        grid=(N, s),
        in_specs=[
            pl.BlockSpec((COUT, CIN), lambda n, t: (0, 0)),
            pl.BlockSpec((1, CIN, tile), lambda n, t: (n, 0, t)),
        ],
        out_specs=pl.BlockSpec((1, COUT, tile), lambda n, t: (n, 0, t)),
        compiler_params=pltpu.CompilerParams(
            dimension_semantics=("parallel", "parallel"),
            vmem_limit_bytes=vmem,
        ),
        cost_estimate=pl.CostEstimate(
            flops=2 * N * HW * CIN * COUT,
            transcendentals=0,
            bytes_accessed=(N * CIN * HW + COUT * CIN + N * COUT * HW) * 4,
        ),
    )(w2d, x3d)
    return out3d.reshape(N, COUT, H, W)
